# baseline (speedup 1.0000x reference)
import functools

import jax
import jax.numpy as jnp
from jax import lax
from jax.experimental import pallas as pl
from jax.experimental.pallas import tpu as pltpu
from jax.experimental.pallas import tpu_sc as plsc

N = 10000
D = 128
E = 320000

NC = 2
NS = 16
NW = NC * NS
CHUNK = 128
CH_PER_TILE = 80
E_PAD = NW * CH_PER_TILE * CHUNK
NDUM = 512
N_PAD = N + NDUM
ROWS_PER_TILE = N_PAD // NS
CNTW = 128

TOTCH = NW * CH_PER_TILE

_SC_MESH = plsc.VectorSubcoreMesh(
    core_axis_name="c", subcore_axis_name="s", num_cores=NC, num_subcores=NS)


@functools.partial(
    pl.kernel,
    out_type=jax.ShapeDtypeStruct((NW, ROWS_PER_TILE, D), jnp.float32),
    mesh=_SC_MESH,
    scratch_types=[
        pltpu.VMEM((CH_PER_TILE, CHUNK), jnp.int32),
        pltpu.VMEM((CH_PER_TILE, CHUNK), jnp.int32),
        pltpu.VMEM((CHUNK, D), jnp.float32),
        pltpu.VMEM_SHARED((N_PAD, D), jnp.float32),
        pltpu.SemaphoreType.DMA,
    ],
)
def _sc_agg(h_hbm, src_hbm, dst_hbm, zero_hbm, out_hbm,
            src_v, dst_v, rows_v, agg_sh, gsem):
    cid = lax.axis_index("c")
    sid = lax.axis_index("s")
    gtid = cid * NS + sid
    pltpu.sync_copy(zero_hbm, agg_sh.at[pl.ds(sid * ROWS_PER_TILE, ROWS_PER_TILE)])
    pltpu.sync_copy(src_hbm.at[gtid], src_v)
    pltpu.sync_copy(dst_hbm.at[gtid], dst_v)
    plsc.subcore_barrier()

    def step(j, carry):
        pltpu.async_copy(h_hbm.at[src_v.at[j]], rows_v, gsem).wait()
        pltpu.sync_copy(rows_v, agg_sh.at[dst_v.at[j]], add=True)
        return carry

    lax.fori_loop(0, CH_PER_TILE, step, 0)
    plsc.subcore_barrier()
    pltpu.sync_copy(agg_sh.at[pl.ds(sid * ROWS_PER_TILE, ROWS_PER_TILE)],
                    out_hbm.at[gtid])


@functools.partial(
    pl.kernel,
    out_type=jax.ShapeDtypeStruct((NW, ROWS_PER_TILE, CNTW), jnp.float32),
    mesh=_SC_MESH,
    scratch_types=[
        pltpu.VMEM((CH_PER_TILE, CHUNK), jnp.int32),
        pltpu.VMEM((CHUNK, CNTW), jnp.float32),
        pltpu.VMEM_SHARED((N_PAD, CNTW), jnp.float32),
    ],
)
def _sc_count(dst_hbm, ones_hbm, zero_hbm, out_hbm, dst_v, ones_v, cnt_sh):
    cid = lax.axis_index("c")
    sid = lax.axis_index("s")
    gtid = cid * NS + sid
    pltpu.sync_copy(zero_hbm, cnt_sh.at[pl.ds(sid * ROWS_PER_TILE, ROWS_PER_TILE)])
    pltpu.sync_copy(dst_hbm.at[gtid], dst_v)
    pltpu.sync_copy(ones_hbm, ones_v)
    plsc.subcore_barrier()

    def step(j, carry):
        pltpu.sync_copy(ones_v, cnt_sh.at[dst_v.at[j]], add=True)
        return carry

    lax.fori_loop(0, CH_PER_TILE, step, 0)
    plsc.subcore_barrier()
    pltpu.sync_copy(cnt_sh.at[pl.ds(sid * ROWS_PER_TILE, ROWS_PER_TILE)],
                    out_hbm.at[gtid])


_BLK = 1000


def _tc_in_body(x_ref, w_ref, b_ref, o_ref):
    o_ref[...] = (jnp.dot(x_ref[...], w_ref[...],
                          preferred_element_type=jnp.float32) + b_ref[...])


def _tc_layer_body(h_ref, g_ref, c_ref, ws_ref, bs_ref, wn_ref, o_ref):
    g = g_ref[0] + g_ref[1]
    cnt = c_ref[0, :, 0:1] + c_ref[1, :, 0:1]
    agg = g / jnp.maximum(cnt, 1.0)
    t = (jnp.dot(h_ref[...], ws_ref[...], preferred_element_type=jnp.float32)
         + jnp.dot(agg, wn_ref[...], preferred_element_type=jnp.float32)
         + bs_ref[...])
    o_ref[...] = jnp.where(t > 0.0, t, jnp.exp(jnp.minimum(t, 0.0)) - 1.0)


def _tc_input_proj(x, w, b):
    return pl.pallas_call(
        _tc_in_body,
        grid=(N // _BLK,),
        in_specs=[
            pl.BlockSpec((_BLK, D), lambda i: (i, 0)),
            pl.BlockSpec((D, D), lambda i: (0, 0)),
            pl.BlockSpec((1, D), lambda i: (0, 0)),
        ],
        out_specs=pl.BlockSpec((_BLK, D), lambda i: (i, 0)),
        out_shape=jax.ShapeDtypeStruct((N, D), jnp.float32),
    )(x, w, b.reshape(1, D))


def _tc_layer(h, g_parts, c_parts, ws, bs, wn):
    return pl.pallas_call(
        _tc_layer_body,
        grid=(N // _BLK,),
        in_specs=[
            pl.BlockSpec((_BLK, D), lambda i: (i, 0)),
            pl.BlockSpec((NC, _BLK, D), lambda i: (0, i, 0)),
            pl.BlockSpec((NC, _BLK, CNTW), lambda i: (0, i, 0)),
            pl.BlockSpec((D, D), lambda i: (0, 0)),
            pl.BlockSpec((1, D), lambda i: (0, 0)),
            pl.BlockSpec((D, D), lambda i: (0, 0)),
        ],
        out_specs=pl.BlockSpec((_BLK, D), lambda i: (i, 0)),
        out_shape=jax.ShapeDtypeStruct((N, D), jnp.float32),
    )(h, g_parts, c_parts, ws, bs.reshape(1, D), wn)


def kernel(x, edge_index, W_in, b_in, Ws0, bs0, Wn0, Ws1, bs1, Wn1, Ws2, bs2, Wn2):
    src = edge_index[0].astype(jnp.int32)
    dst = edge_index[1].astype(jnp.int32)
    pad = E_PAD - E
    src_p = jnp.concatenate([src, jnp.zeros((pad,), jnp.int32)])
    dst_p = jnp.concatenate(
        [dst, N + (jnp.arange(pad, dtype=jnp.int32) % NDUM)])
    src_p = src_p.reshape(TOTCH, CHUNK)
    dst_p = dst_p.reshape(TOTCH, CHUNK)
    src_t = src_p.reshape(NW, CH_PER_TILE, CHUNK)
    dst_t = dst_p.reshape(NW, CH_PER_TILE, CHUNK)
    dst_pc = dst_t

    zero_rows = jnp.zeros((ROWS_PER_TILE, D), jnp.float32)
    zero_cnt = jnp.zeros((ROWS_PER_TILE, CNTW), jnp.float32)
    ones_rows = jnp.ones((CHUNK, CNTW), jnp.float32)

    c_parts = _sc_count(dst_pc, ones_rows, zero_cnt).reshape(NC, N_PAD, CNTW)

    h = _tc_input_proj(x, W_in, b_in)
    for ws, bs, wn in ((Ws0, bs0, Wn0), (Ws1, bs1, Wn1), (Ws2, bs2, Wn2)):
        g_parts = _sc_agg(h, src_t, dst_t, zero_rows).reshape(NC, N_PAD, D)
        h = _tc_layer(h, g_parts, c_parts, ws, bs, wn)
    return h


# speedup vs baseline: 1.5169x; 1.5169x over previous
import functools

import jax
import jax.numpy as jnp
from jax import lax
from jax.experimental import pallas as pl
from jax.experimental.pallas import tpu as pltpu
from jax.experimental.pallas import tpu_sc as plsc

N = 10000
D = 128
E = 320000

NC = 2
NS = 16
NW = NC * NS
CHUNK = 128
CH_PER_TILE = 79
E_PAD = NW * CH_PER_TILE * CHUNK
NDUM = 512
N_PAD = N + NDUM
ROWS_PER_TILE = N_PAD // NS
CNTW = 128

TOTCH = NW * CH_PER_TILE

_SC_MESH = plsc.VectorSubcoreMesh(
    core_axis_name="c", subcore_axis_name="s", num_cores=NC, num_subcores=NS)


@functools.partial(
    pl.kernel,
    out_type=jax.ShapeDtypeStruct((NW, ROWS_PER_TILE, D), jnp.float32),
    mesh=_SC_MESH,
    scratch_types=[
        pltpu.VMEM((CH_PER_TILE, CHUNK), jnp.int32),
        pltpu.VMEM((CH_PER_TILE, CHUNK), jnp.int32),
        pltpu.VMEM((CHUNK, D), jnp.float32),
        pltpu.VMEM_SHARED((N_PAD, D), jnp.float32),
        pltpu.SemaphoreType.DMA,
    ],
)
def _sc_agg(h_hbm, src_hbm, dst_hbm, zero_hbm, out_hbm,
            src_v, dst_v, rows_v, agg_sh, gsem):
    cid = lax.axis_index("c")
    sid = lax.axis_index("s")
    gtid = cid * NS + sid
    pltpu.sync_copy(zero_hbm, agg_sh.at[pl.ds(sid * ROWS_PER_TILE, ROWS_PER_TILE)])
    pltpu.sync_copy(src_hbm.at[gtid], src_v)
    pltpu.sync_copy(dst_hbm.at[gtid], dst_v)
    plsc.subcore_barrier()

    def step(j, carry):
        pltpu.async_copy(h_hbm.at[src_v.at[j]], rows_v, gsem).wait()
        pltpu.sync_copy(rows_v, agg_sh.at[dst_v.at[j]], add=True)
        return carry

    lax.fori_loop(0, CH_PER_TILE, step, 0)
    plsc.subcore_barrier()
    pltpu.sync_copy(agg_sh.at[pl.ds(sid * ROWS_PER_TILE, ROWS_PER_TILE)],
                    out_hbm.at[gtid])


@functools.partial(
    pl.kernel,
    out_type=jax.ShapeDtypeStruct((NW, ROWS_PER_TILE, CNTW), jnp.float32),
    mesh=_SC_MESH,
    scratch_types=[
        pltpu.VMEM((CH_PER_TILE, CHUNK), jnp.int32),
        pltpu.VMEM((CHUNK, CNTW), jnp.float32),
        pltpu.VMEM_SHARED((N_PAD, CNTW), jnp.float32),
    ],
)
def _sc_count(dst_hbm, ones_hbm, zero_hbm, out_hbm, dst_v, ones_v, cnt_sh):
    cid = lax.axis_index("c")
    sid = lax.axis_index("s")
    gtid = cid * NS + sid
    pltpu.sync_copy(zero_hbm, cnt_sh.at[pl.ds(sid * ROWS_PER_TILE, ROWS_PER_TILE)])
    pltpu.sync_copy(dst_hbm.at[gtid], dst_v)
    pltpu.sync_copy(ones_hbm, ones_v)
    plsc.subcore_barrier()

    def step(j, carry):
        pltpu.sync_copy(ones_v, cnt_sh.at[dst_v.at[j]], add=True)
        return carry

    lax.fori_loop(0, CH_PER_TILE, step, 0)
    plsc.subcore_barrier()
    pltpu.sync_copy(cnt_sh.at[pl.ds(sid * ROWS_PER_TILE, ROWS_PER_TILE)],
                    out_hbm.at[gtid])


_BLK = 1000


def _tc_in_body(x_ref, w_ref, b_ref, o_ref):
    o_ref[...] = (jnp.dot(x_ref[...], w_ref[...],
                          preferred_element_type=jnp.float32) + b_ref[...])


def _tc_layer_body(h_ref, g_ref, c_ref, ws_ref, bs_ref, wn_ref, o_ref):
    g = g_ref[0] + g_ref[1]
    cnt = c_ref[0, :, 0:1] + c_ref[1, :, 0:1]
    agg = g / jnp.maximum(cnt, 1.0)
    t = (jnp.dot(h_ref[...], ws_ref[...], preferred_element_type=jnp.float32)
         + jnp.dot(agg, wn_ref[...], preferred_element_type=jnp.float32)
         + bs_ref[...])
    o_ref[...] = jnp.where(t > 0.0, t, jnp.exp(jnp.minimum(t, 0.0)) - 1.0)


def _tc_input_proj(x, w, b):
    return pl.pallas_call(
        _tc_in_body,
        grid=(N // _BLK,),
        in_specs=[
            pl.BlockSpec((_BLK, D), lambda i: (i, 0)),
            pl.BlockSpec((D, D), lambda i: (0, 0)),
            pl.BlockSpec((1, D), lambda i: (0, 0)),
        ],
        out_specs=pl.BlockSpec((_BLK, D), lambda i: (i, 0)),
        out_shape=jax.ShapeDtypeStruct((N, D), jnp.float32),
    )(x, w, b.reshape(1, D))


def _tc_layer(h, g_parts, c_parts, ws, bs, wn):
    return pl.pallas_call(
        _tc_layer_body,
        grid=(N // _BLK,),
        in_specs=[
            pl.BlockSpec((_BLK, D), lambda i: (i, 0)),
            pl.BlockSpec((NC, _BLK, D), lambda i: (0, i, 0)),
            pl.BlockSpec((NC, _BLK, CNTW), lambda i: (0, i, 0)),
            pl.BlockSpec((D, D), lambda i: (0, 0)),
            pl.BlockSpec((1, D), lambda i: (0, 0)),
            pl.BlockSpec((D, D), lambda i: (0, 0)),
        ],
        out_specs=pl.BlockSpec((_BLK, D), lambda i: (i, 0)),
        out_shape=jax.ShapeDtypeStruct((N, D), jnp.float32),
    )(h, g_parts, c_parts, ws, bs.reshape(1, D), wn)


def kernel(x, edge_index, W_in, b_in, Ws0, bs0, Wn0, Ws1, bs1, Wn1, Ws2, bs2, Wn2):
    src = edge_index[0].astype(jnp.int32)
    dst = edge_index[1].astype(jnp.int32)
    pad = E_PAD - E
    src_p = jnp.concatenate([src, jnp.zeros((pad,), jnp.int32)])
    dst_p = jnp.concatenate(
        [dst, N + (jnp.arange(pad, dtype=jnp.int32) % NDUM)])
    src_p = src_p.reshape(TOTCH, CHUNK)
    dst_p = dst_p.reshape(TOTCH, CHUNK)
    src_t = src_p.reshape(NW, CH_PER_TILE, CHUNK)
    dst_t = dst_p.reshape(NW, CH_PER_TILE, CHUNK)
    dst_pc = dst_t

    zero_rows = jnp.zeros((ROWS_PER_TILE, D), jnp.float32)
    zero_cnt = jnp.zeros((ROWS_PER_TILE, CNTW), jnp.float32)
    ones_rows = jnp.ones((CHUNK, CNTW), jnp.float32)

    c_parts = _sc_count(dst_pc, ones_rows, zero_cnt).reshape(NC, N_PAD, CNTW)

    h = _tc_input_proj(x, W_in, b_in)
    for ws, bs, wn in ((Ws0, bs0, Wn0), (Ws1, bs1, Wn1), (Ws2, bs2, Wn2)):
        g_parts = _sc_agg(h, src_t, dst_t, zero_rows).reshape(NC, N_PAD, D)
        h = _tc_layer(h, g_parts, c_parts, ws, bs, wn)
    return h
